# baseline (speedup 1.0000x reference)
import functools

import jax
import jax.numpy as jnp
from jax import lax
from jax.experimental import pallas as pl
from jax.experimental.pallas import tpu as pltpu
from jax.experimental.pallas import tpu_sc as plsc

D = 64
H = 4
LANES = 16
NC = 2
NS = 16
NW = NC * NS
CHUNK = 128
BPC = CHUNK // H


def _sc_pool_body(ids_hbm, table_hbm, out_hbm, idx_v, rows_v, out_v, sem):
    c = lax.axis_index("c")
    s = lax.axis_index("s")
    wid = s * NC + c
    nchunk = ids_hbm.shape[1]
    batch_per_w = nchunk * BPC
    pltpu.sync_copy(ids_hbm.at[wid], idx_v)

    def chunk_body(ci, _):
        pltpu.async_copy(table_hbm.at[idx_v.at[ci]], rows_v, sem).wait()

        def red_body(b, __):
            row = ci * BPC + b
            for col in range(D // LANES):
                sl = pl.ds(col * LANES, LANES)
                v = (rows_v[4 * b, sl] + rows_v[4 * b + 1, sl]
                     + rows_v[4 * b + 2, sl] + rows_v[4 * b + 3, sl])
                out_v[row, sl] = v
            return __

        lax.fori_loop(0, BPC, red_body, 0, unroll=False)
        return _

    lax.fori_loop(0, nchunk, chunk_body, 0, unroll=False)
    pltpu.sync_copy(out_v, out_hbm.at[pl.ds(wid * batch_per_w, batch_per_w)])


def _sc_pool(ids_r, table):
    nw, nchunk, chunk = ids_r.shape
    batch_per_w = nchunk * BPC
    batch = nw * batch_per_w
    mesh = plsc.VectorSubcoreMesh(core_axis_name="c", subcore_axis_name="s")
    kern = pl.kernel(
        _sc_pool_body,
        mesh=mesh,
        compiler_params=pltpu.CompilerParams(use_tc_tiling_on_sc=False),
        out_type=jax.ShapeDtypeStruct((batch, D), jnp.float32),
        scratch_types=[
            pltpu.VMEM((nchunk, chunk), jnp.int32),
            pltpu.VMEM((chunk, D), jnp.float32),
            pltpu.VMEM((batch_per_w, D), jnp.float32),
            pltpu.SemaphoreType.DMA,
        ],
    )
    return kern(ids_r, table)


def _tc_proj_body(x_ref, wt_ref, b_ref, o_ref):
    acc = jnp.dot(x_ref[...], wt_ref[...], preferred_element_type=jnp.float32)
    o_ref[...] = jnp.maximum(acc + b_ref[...], 0.0)


def _tc_proj(x, wt, bias2d):
    batch = x.shape[0]
    bm = 2048
    return pl.pallas_call(
        _tc_proj_body,
        grid=(batch // bm,),
        in_specs=[
            pl.BlockSpec((bm, D), lambda i: (i, 0)),
            pl.BlockSpec((D, D), lambda i: (0, 0)),
            pl.BlockSpec((1, D), lambda i: (0, 0)),
        ],
        out_specs=pl.BlockSpec((bm, D), lambda i: (i, 0)),
        out_shape=jax.ShapeDtypeStruct((batch, D), jnp.float32),
    )(x, wt, bias2d)


def kernel(ids, emb_table, proj_w, proj_b):
    batch = ids.shape[0]
    ids_per_w = batch * H // NW
    ids_r = ids.reshape(NW, ids_per_w // CHUNK, CHUNK)
    pooled = _sc_pool(ids_r, emb_table)
    wt = proj_w.T * (1.0 / H)
    return _tc_proj(pooled, wt, proj_b.reshape(1, D))


# speedup vs baseline: 1.5641x; 1.5641x over previous
import functools

import jax
import jax.numpy as jnp
from jax import lax
from jax.experimental import pallas as pl
from jax.experimental.pallas import tpu as pltpu
from jax.experimental.pallas import tpu_sc as plsc

D = 64
H = 4
LANES = 16
NC = 2
NS = 16
NW = NC * NS
CHUNK = 128
BPC = CHUNK // H


def _sc_pool_body(ids_hbm, table_hbm, out_hbm, idx_v, rows_v, out_v, sem, sem_i):
    c = lax.axis_index("c")
    s = lax.axis_index("s")
    wid = s * NC + c
    nchunk = ids_hbm.shape[1]
    batch_per_w = nchunk * BPC

    def chunk_body(ci, _):
        pltpu.sync_copy(ids_hbm.at[wid, ci], idx_v)

        def fire_body(g, __):
            vec = idx_v[pl.ds(g * LANES, LANES)]
            for j in range(LANES):
                r = vec[j]
                pltpu.async_copy(table_hbm.at[r], rows_v.at[g * LANES + j], sem)
            return __

        lax.fori_loop(0, CHUNK // LANES, fire_body, 0, unroll=False)

        def drain_body(i, __):
            pltpu.make_async_copy(table_hbm.at[0], rows_v.at[i], sem).wait()
            return __

        lax.fori_loop(0, CHUNK, drain_body, 0, unroll=False)

        def red_body(b, __):
            row = ci * BPC + b
            for col in range(D // LANES):
                sl = pl.ds(col * LANES, LANES)
                v = (rows_v[4 * b, sl] + rows_v[4 * b + 1, sl]
                     + rows_v[4 * b + 2, sl] + rows_v[4 * b + 3, sl])
                out_v[row, sl] = v
            return __

        lax.fori_loop(0, BPC, red_body, 0, unroll=False)
        return _

    lax.fori_loop(0, nchunk, chunk_body, 0, unroll=False)
    pltpu.sync_copy(out_v, out_hbm.at[pl.ds(wid * batch_per_w, batch_per_w)])


def _sc_pool(ids_r, table):
    nw, nchunk, chunk = ids_r.shape
    batch_per_w = nchunk * BPC
    batch = nw * batch_per_w
    mesh = plsc.VectorSubcoreMesh(core_axis_name="c", subcore_axis_name="s")
    kern = pl.kernel(
        _sc_pool_body,
        mesh=mesh,
        out_type=jax.ShapeDtypeStruct((batch, D), jnp.float32),
        scratch_types=[
            pltpu.VMEM((chunk,), jnp.int32),
            pltpu.VMEM((chunk, D), jnp.float32),
            pltpu.VMEM((batch_per_w, D), jnp.float32),
            pltpu.SemaphoreType.DMA,
            pltpu.SemaphoreType.DMA,
        ],
    )
    return kern(ids_r, table)


def _tc_proj_body(x_ref, wt_ref, b_ref, o_ref):
    acc = jnp.dot(x_ref[...], wt_ref[...], preferred_element_type=jnp.float32)
    o_ref[...] = jnp.maximum(acc + b_ref[...], 0.0)


def _tc_proj(x, wt, bias2d):
    batch = x.shape[0]
    bm = 2048
    return pl.pallas_call(
        _tc_proj_body,
        grid=(batch // bm,),
        in_specs=[
            pl.BlockSpec((bm, D), lambda i: (i, 0)),
            pl.BlockSpec((D, D), lambda i: (0, 0)),
            pl.BlockSpec((1, D), lambda i: (0, 0)),
        ],
        out_specs=pl.BlockSpec((bm, D), lambda i: (i, 0)),
        out_shape=jax.ShapeDtypeStruct((batch, D), jnp.float32),
    )(x, wt, bias2d)


def kernel(ids, emb_table, proj_w, proj_b):
    batch = ids.shape[0]
    ids_per_w = batch * H // NW
    ids_r = ids.reshape(NW, ids_per_w // CHUNK, CHUNK)
    pooled = _sc_pool(ids_r, emb_table)
    wt = proj_w.T * (1.0 / H)
    return _tc_proj(pooled, wt, proj_b.reshape(1, D))


# speedup vs baseline: 1.6515x; 1.0559x over previous
import functools

import jax
import jax.numpy as jnp
from jax import lax
from jax.experimental import pallas as pl
from jax.experimental.pallas import tpu as pltpu
from jax.experimental.pallas import tpu_sc as plsc

D = 64
H = 4
LANES = 16
NC = 2
NS = 16
NW = NC * NS
CHUNK = 128
BPC = CHUNK // H


def _sc_pool_body(ids_hbm, table_hbm, out_hbm, idx_v, rows_a, rows_b, out_v,
                  sem_a, sem_b):
    c = lax.axis_index("c")
    s = lax.axis_index("s")
    wid = s * NC + c
    nchunk = ids_hbm.shape[1]
    batch_per_w = nchunk * BPC

    pltpu.sync_copy(ids_hbm.at[wid], idx_v)

    def fire(ci, buf, sem):
        def group(g, _):
            vec = idx_v[ci, pl.ds(g * LANES, LANES)]
            for j in range(LANES):
                pltpu.async_copy(table_hbm.at[vec[j]], buf.at[g * LANES + j],
                                 sem)
            return _

        lax.fori_loop(0, CHUNK // LANES, group, 0, unroll=False)

    def drain(buf, sem):
        pltpu.make_async_copy(table_hbm.at[pl.ds(0, CHUNK)], buf, sem).wait()

    def reduce(ci, buf):
        def body(b, _):
            row = ci * BPC + b
            for col in range(D // LANES):
                sl = pl.ds(col * LANES, LANES)
                v = (buf[4 * b, sl] + buf[4 * b + 1, sl]
                     + buf[4 * b + 2, sl] + buf[4 * b + 3, sl])
                out_v[row, sl] = v
            return _

        lax.fori_loop(0, BPC, body, 0, unroll=False)

    fire(0, rows_a, sem_a)

    def pair_body(i, _):
        ca = 2 * i
        fire(ca + 1, rows_b, sem_b)
        drain(rows_a, sem_a)
        reduce(ca, rows_a)

        @pl.when(i < nchunk // 2 - 1)
        def _fire_next():
            fire(ca + 2, rows_a, sem_a)

        drain(rows_b, sem_b)
        reduce(ca + 1, rows_b)
        return _

    lax.fori_loop(0, nchunk // 2, pair_body, 0, unroll=False)
    pltpu.sync_copy(out_v, out_hbm.at[pl.ds(wid * batch_per_w, batch_per_w)])


def _sc_pool(ids_r, table):
    nw, nchunk, chunk = ids_r.shape
    batch_per_w = nchunk * BPC
    batch = nw * batch_per_w
    mesh = plsc.VectorSubcoreMesh(core_axis_name="c", subcore_axis_name="s")
    kern = pl.kernel(
        _sc_pool_body,
        mesh=mesh,
        out_type=jax.ShapeDtypeStruct((batch, D), jnp.float32),
        scratch_types=[
            pltpu.VMEM((nchunk, chunk), jnp.int32),
            pltpu.VMEM((chunk, D), jnp.float32),
            pltpu.VMEM((chunk, D), jnp.float32),
            pltpu.VMEM((batch_per_w, D), jnp.float32),
            pltpu.SemaphoreType.DMA,
            pltpu.SemaphoreType.DMA,
        ],
    )
    return kern(ids_r, table)


def _tc_proj_body(x_ref, wt_ref, b_ref, o_ref):
    acc = jnp.dot(x_ref[...], wt_ref[...], preferred_element_type=jnp.float32)
    o_ref[...] = jnp.maximum(acc + b_ref[...], 0.0)


def _tc_proj(x, wt, bias2d):
    batch = x.shape[0]
    bm = 2048
    return pl.pallas_call(
        _tc_proj_body,
        grid=(batch // bm,),
        in_specs=[
            pl.BlockSpec((bm, D), lambda i: (i, 0)),
            pl.BlockSpec((D, D), lambda i: (0, 0)),
            pl.BlockSpec((1, D), lambda i: (0, 0)),
        ],
        out_specs=pl.BlockSpec((bm, D), lambda i: (i, 0)),
        out_shape=jax.ShapeDtypeStruct((batch, D), jnp.float32),
    )(x, wt, bias2d)


def kernel(ids, emb_table, proj_w, proj_b):
    batch = ids.shape[0]
    ids_per_w = batch * H // NW
    ids_r = ids.reshape(NW, ids_per_w // CHUNK, CHUNK)
    pooled = _sc_pool(ids_r, emb_table)
    wt = proj_w.T * (1.0 / H)
    return _tc_proj(pooled, wt, proj_b.reshape(1, D))


# speedup vs baseline: 2.3244x; 1.4074x over previous
import functools

import jax
import jax.numpy as jnp
from jax import lax
from jax.experimental import pallas as pl
from jax.experimental.pallas import tpu as pltpu
from jax.experimental.pallas import tpu_sc as plsc

D = 64
H = 4
LANES = 16
NC = 2
NS = 16
NW = NC * NS
CHUNK = 128
BPC = CHUNK // H
VBLK = 8192
SPLIT = 62 * VBLK


def _tc_pack_body(t_lo_ref, t_hi_ref, wt_ref, o_ref):
    dn = (((0,), (0,)), ((), ()))
    o_ref[:, 0:D] = lax.dot_general(t_lo_ref[...], wt_ref[...], dn,
                                    preferred_element_type=jnp.float32)
    o_ref[:, D:2 * D] = lax.dot_general(t_hi_ref[...], wt_ref[...], dn,
                                        preferred_element_type=jnp.float32)


def _tc_pack(table_t, wt):
    return pl.pallas_call(
        _tc_pack_body,
        grid=(SPLIT // VBLK,),
        in_specs=[
            pl.BlockSpec((D, VBLK), lambda i: (0, i)),
            pl.BlockSpec((D, VBLK),
                         lambda i: (0, jnp.minimum(i + SPLIT // VBLK,
                                                   1000000 // VBLK))),
            pl.BlockSpec((D, D), lambda i: (0, 0)),
        ],
        out_specs=pl.BlockSpec((VBLK, 2 * D), lambda i: (i, 0)),
        out_shape=jax.ShapeDtypeStruct((SPLIT, 2 * D), jnp.float32),
    )(table_t, table_t, wt)


def _sc_pool_body(ids_hbm, p_hbm, bias_hbm, out_hbm, idx_v, off_v, bias_v,
                  rows_a, rows_b, out_v, sem_a, sem_b):
    c = lax.axis_index("c")
    s = lax.axis_index("s")
    wid = s * NC + c
    nchunk = ids_hbm.shape[1]
    batch_per_w = nchunk * BPC

    pltpu.sync_copy(ids_hbm.at[wid], idx_v)
    pltpu.sync_copy(bias_hbm, bias_v)

    def prep(ci, _):
        def grp(g, __):
            sl = pl.ds(g * LANES, LANES)
            v = idx_v[ci, sl]
            hi = v >= SPLIT
            idx_v[ci, sl] = jnp.where(hi, v - SPLIT, v)
            off_v[ci, sl] = jnp.where(hi, D, 0)
            return __
        lax.fori_loop(0, CHUNK // LANES, grp, 0, unroll=False)
        return _

    lax.fori_loop(0, nchunk, prep, 0, unroll=False)

    bias_regs = [bias_v[pl.ds(cc * LANES, LANES)] for cc in range(D // LANES)]

    def fire(ci, buf, sem):
        pltpu.async_copy(p_hbm.at[idx_v.at[ci]], buf, sem)

    def drain(buf, sem):
        pltpu.make_async_copy(p_hbm.at[pl.ds(0, CHUNK)], buf, sem).wait()

    def pool(ci, buf):
        def grp(g, _):
            off_vec = off_v[ci, pl.ds(g * LANES, LANES)]
            for bb in range(LANES // H):
                b = g * (LANES // H) + bb
                offs = [off_vec[H * bb + j] for j in range(H)]
                for cc in range(D // LANES):
                    v = bias_regs[cc]
                    for j in range(H):
                        v = v + buf[g * LANES + H * bb + j,
                                    pl.ds(offs[j] + cc * LANES, LANES)]
                    out_v[ci * BPC + b, pl.ds(cc * LANES, LANES)] = (
                        jnp.maximum(v, 0.0))
            return _

        lax.fori_loop(0, CHUNK // LANES, grp, 0, unroll=False)

    fire(0, rows_a, sem_a)

    def pair_body(i, _):
        ca = 2 * i
        fire(ca + 1, rows_b, sem_b)
        drain(rows_a, sem_a)
        pool(ca, rows_a)

        @pl.when(i < nchunk // 2 - 1)
        def _fire_next():
            fire(ca + 2, rows_a, sem_a)

        drain(rows_b, sem_b)
        pool(ca + 1, rows_b)
        return _

    lax.fori_loop(0, nchunk // 2, pair_body, 0, unroll=False)
    pltpu.sync_copy(out_v, out_hbm.at[pl.ds(wid * batch_per_w, batch_per_w)])


def _sc_pool(ids_r, p_packed, bias):
    nw, nchunk, chunk = ids_r.shape
    batch_per_w = nchunk * BPC
    batch = nw * batch_per_w
    mesh = plsc.VectorSubcoreMesh(core_axis_name="c", subcore_axis_name="s")
    kern = pl.kernel(
        _sc_pool_body,
        mesh=mesh,
        out_type=jax.ShapeDtypeStruct((batch, D), jnp.float32),
        scratch_types=[
            pltpu.VMEM((nchunk, chunk), jnp.int32),
            pltpu.VMEM((nchunk, chunk), jnp.int32),
            pltpu.VMEM((D,), jnp.float32),
            pltpu.VMEM((chunk, 2 * D), jnp.float32),
            pltpu.VMEM((chunk, 2 * D), jnp.float32),
            pltpu.VMEM((batch_per_w, D), jnp.float32),
            pltpu.SemaphoreType.DMA,
            pltpu.SemaphoreType.DMA,
        ],
    )
    return kern(ids_r, p_packed, bias)


def kernel(ids, emb_table, proj_w, proj_b):
    batch = ids.shape[0]
    ids_per_w = batch * H // NW
    ids_r = ids.reshape(NW, ids_per_w // CHUNK, CHUNK)
    wt = proj_w.T * (1.0 / H)
    p_packed = _tc_pack(emb_table.T, wt)
    return _sc_pool(ids_r, p_packed, proj_b)


# speedup vs baseline: 2.7320x; 1.1754x over previous
import functools

import jax
import jax.numpy as jnp
from jax import lax
from jax.experimental import pallas as pl
from jax.experimental.pallas import tpu as pltpu
from jax.experimental.pallas import tpu_sc as plsc

D = 64
H = 4
LANES = 16
NC = 2
NS = 16
NW = NC * NS
CHUNK = 128
BPC = CHUNK // H
VBLK = 8192
NBLK = 31
S2 = NBLK * VBLK
VOCAB = 1000000
LAST_BLK = VOCAB // VBLK


def _pack_pair(a, b):
    ua = lax.bitcast_convert_type(a, jnp.int32)
    ub = lax.bitcast_convert_type(b, jnp.int32)
    hi = (ua + 0x8000) & jnp.int32(-65536)
    lo = lax.shift_right_logical(ub + 0x8000, 16)
    return lax.bitcast_convert_type(hi | lo, jnp.float32)


def _tc_pack_body(t0_ref, t1_ref, t2_ref, t3_ref, wt_ref, o_ref):
    dn = (((0,), (0,)), ((), ()))
    wt = wt_ref[...]
    ys = [lax.dot_general(t_ref[...], wt, dn,
                          preferred_element_type=jnp.float32)
          for t_ref in (t0_ref, t1_ref, t2_ref, t3_ref)]
    o_ref[...] = jnp.concatenate(
        [_pack_pair(ys[0], ys[1]), _pack_pair(ys[2], ys[3])], axis=1)


def _tc_pack(table_t, wt):
    def spec(q):
        return pl.BlockSpec(
            (D, VBLK),
            lambda i, q=q: (0, jnp.minimum(i + q * NBLK, LAST_BLK)))

    return pl.pallas_call(
        _tc_pack_body,
        grid=(NBLK,),
        in_specs=[spec(0), spec(1), spec(2), spec(3),
                  pl.BlockSpec((D, D), lambda i: (0, 0))],
        out_specs=pl.BlockSpec((VBLK, 2 * D), lambda i: (i, 0)),
        out_shape=jax.ShapeDtypeStruct((S2, 2 * D), jnp.float32),
    )(table_t, table_t, table_t, table_t, wt)


def _sc_pool_body(ids_hbm, p_hbm, bias_hbm, out_hbm, idx_v, off_v, mul_v,
                  bias_v, rows_a, rows_b, out_v, sem_a, sem_b):
    c = lax.axis_index("c")
    s = lax.axis_index("s")
    wid = s * NC + c
    nchunk = ids_hbm.shape[1]
    batch_per_w = nchunk * BPC

    pltpu.sync_copy(ids_hbm.at[wid], idx_v)
    pltpu.sync_copy(bias_hbm, bias_v)

    def prep(ci, _):
        def grp(g, __):
            sl = pl.ds(g * LANES, LANES)
            v = idx_v[ci, sl]
            zero = jnp.zeros((LANES,), jnp.int32)
            q = (jnp.where(v >= S2, 1, zero) + jnp.where(v >= 2 * S2, 1, zero)
                 + jnp.where(v >= 3 * S2, 1, zero))
            idx_v[ci, sl] = v - q * S2
            off_v[ci, sl] = jnp.where(q >= 2, D, zero)
            mul_v[ci, sl] = jnp.where((q & 1) > 0, 65536, 1 + zero)
            return __
        lax.fori_loop(0, CHUNK // LANES, grp, 0, unroll=False)
        return _

    lax.fori_loop(0, nchunk, prep, 0, unroll=False)

    bias_regs = [bias_v[pl.ds(cc * LANES, LANES)] for cc in range(D // LANES)]

    def fire(ci, buf, sem):
        pltpu.async_copy(p_hbm.at[idx_v.at[ci]], buf, sem)

    def drain(buf, sem):
        pltpu.make_async_copy(p_hbm.at[pl.ds(0, CHUNK)], buf, sem).wait()

    def pool(ci, buf):
        def grp(g, _):
            off_vec = off_v[ci, pl.ds(g * LANES, LANES)]
            mul_vec = mul_v[ci, pl.ds(g * LANES, LANES)]
            for bb in range(LANES // H):
                b = g * (LANES // H) + bb
                offs = [off_vec[H * bb + j] for j in range(H)]
                muls = [mul_vec[H * bb + j] for j in range(H)]
                for cc in range(D // LANES):
                    v = bias_regs[cc]
                    for j in range(H):
                        w = lax.bitcast_convert_type(
                            buf[g * LANES + H * bb + j,
                                pl.ds(offs[j] + cc * LANES, LANES)], jnp.int32)
                        v = v + lax.bitcast_convert_type(
                            (w * muls[j]) & jnp.int32(-65536), jnp.float32)
                    out_v[ci * BPC + b, pl.ds(cc * LANES, LANES)] = (
                        jnp.maximum(v, 0.0))
            return _

        lax.fori_loop(0, CHUNK // LANES, grp, 0, unroll=False)

    fire(0, rows_a, sem_a)

    def pair_body(i, _):
        ca = 2 * i
        fire(ca + 1, rows_b, sem_b)
        drain(rows_a, sem_a)
        pool(ca, rows_a)

        @pl.when(i < nchunk // 2 - 1)
        def _fire_next():
            fire(ca + 2, rows_a, sem_a)

        drain(rows_b, sem_b)
        pool(ca + 1, rows_b)
        return _

    lax.fori_loop(0, nchunk // 2, pair_body, 0, unroll=False)
    pltpu.sync_copy(out_v, out_hbm.at[pl.ds(wid * batch_per_w, batch_per_w)])


def _sc_pool(ids_r, p_packed, bias):
    nw, nchunk, chunk = ids_r.shape
    batch_per_w = nchunk * BPC
    batch = nw * batch_per_w
    mesh = plsc.VectorSubcoreMesh(core_axis_name="c", subcore_axis_name="s")
    kern = pl.kernel(
        _sc_pool_body,
        mesh=mesh,
        out_type=jax.ShapeDtypeStruct((batch, D), jnp.float32),
        scratch_types=[
            pltpu.VMEM((nchunk, chunk), jnp.int32),
            pltpu.VMEM((nchunk, chunk), jnp.int32),
            pltpu.VMEM((nchunk, chunk), jnp.int32),
            pltpu.VMEM((D,), jnp.float32),
            pltpu.VMEM((chunk, 2 * D), jnp.float32),
            pltpu.VMEM((chunk, 2 * D), jnp.float32),
            pltpu.VMEM((batch_per_w, D), jnp.float32),
            pltpu.SemaphoreType.DMA,
            pltpu.SemaphoreType.DMA,
        ],
    )
    return kern(ids_r, p_packed, bias)


def kernel(ids, emb_table, proj_w, proj_b):
    batch = ids.shape[0]
    ids_per_w = batch * H // NW
    ids_r = ids.reshape(NW, ids_per_w // CHUNK, CHUNK)
    wt = proj_w.T * (1.0 / H)
    p_packed = _tc_pack(emb_table.T, wt)
    return _sc_pool(ids_r, p_packed, proj_b)


# speedup vs baseline: 3.0888x; 1.1306x over previous
import functools

import jax
import jax.numpy as jnp
from jax import lax
from jax.experimental import pallas as pl
from jax.experimental.pallas import tpu as pltpu
from jax.experimental.pallas import tpu_sc as plsc

D = 64
H = 4
LANES = 16
NC = 2
NS = 16
NW = NC * NS
CHUNK = 128
BPC = CHUNK // H
VBLK = 8192
NBLK = 31
S2 = NBLK * VBLK
VOCAB = 1000000
LAST_BLK = VOCAB // VBLK


def _pack_pair(a, b):
    ua = lax.bitcast_convert_type(a, jnp.int32)
    ub = lax.bitcast_convert_type(b, jnp.int32)
    hi = (ua + 0x8000) & jnp.int32(-65536)
    lo = lax.shift_right_logical(ub + 0x8000, 16)
    return lax.bitcast_convert_type(hi | lo, jnp.float32)


def _tc_pack_body(t0_ref, t1_ref, t2_ref, t3_ref, wt_ref, o_ref):
    dn = (((0,), (0,)), ((), ()))
    wt = wt_ref[...]
    ys = [lax.dot_general(wt, t_ref[...], dn,
                          preferred_element_type=jnp.float32)
          for t_ref in (t0_ref, t1_ref, t2_ref, t3_ref)]
    p01 = jnp.transpose(_pack_pair(ys[0], ys[1]))
    p23 = jnp.transpose(_pack_pair(ys[2], ys[3]))
    o_ref[...] = jnp.concatenate([p01, p23], axis=1)


def _tc_pack(table_t, wt):
    def spec(q):
        return pl.BlockSpec(
            (D, VBLK),
            lambda i, q=q: (0, jnp.minimum(i + q * NBLK, LAST_BLK)))

    return pl.pallas_call(
        _tc_pack_body,
        grid=(NBLK,),
        in_specs=[spec(0), spec(1), spec(2), spec(3),
                  pl.BlockSpec((D, D), lambda i: (0, 0))],
        out_specs=pl.BlockSpec((VBLK, 2 * D), lambda i: (i, 0)),
        out_shape=jax.ShapeDtypeStruct((S2, 2 * D), jnp.float32),
    )(table_t, table_t, table_t, table_t, wt)


def _sc_pool_body(ids_hbm, p_hbm, bias_hbm, out_hbm, idx_v, off_v, mul_v,
                  bias_v, rows_a, rows_b, out_v, sem_a, sem_b):
    c = lax.axis_index("c")
    s = lax.axis_index("s")
    wid = s * NC + c
    nchunk = ids_hbm.shape[1]
    batch_per_w = nchunk * BPC

    pltpu.sync_copy(ids_hbm.at[wid], idx_v)
    pltpu.sync_copy(bias_hbm, bias_v)

    def prep(ci, _):
        def grp(g, __):
            sl = pl.ds(g * LANES, LANES)
            v = idx_v[ci, sl]
            zero = jnp.zeros((LANES,), jnp.int32)
            q = (jnp.where(v >= S2, 1, zero) + jnp.where(v >= 2 * S2, 1, zero)
                 + jnp.where(v >= 3 * S2, 1, zero))
            idx_v[ci, sl] = v - q * S2
            off_v[ci, sl] = jnp.where(q >= 2, D, zero)
            mul_v[ci, sl] = jnp.where((q & 1) > 0, 65536, 1 + zero)
            return __
        lax.fori_loop(0, CHUNK // LANES, grp, 0, unroll=False)
        return _

    lax.fori_loop(0, nchunk, prep, 0, unroll=False)

    bias_regs = [bias_v[pl.ds(cc * LANES, LANES)] for cc in range(D // LANES)]

    def fire(ci, buf, sem):
        pltpu.async_copy(p_hbm.at[idx_v.at[ci]], buf, sem)

    def drain(buf, sem):
        pltpu.make_async_copy(p_hbm.at[pl.ds(0, CHUNK)], buf, sem).wait()

    def pool(ci, buf):
        def grp(g, _):
            off_vec = off_v[ci, pl.ds(g * LANES, LANES)]
            mul_vec = mul_v[ci, pl.ds(g * LANES, LANES)]
            for bb in range(LANES // H):
                b = g * (LANES // H) + bb
                offs = [off_vec[H * bb + j] for j in range(H)]
                muls = [mul_vec[H * bb + j] for j in range(H)]
                for cc in range(D // LANES):
                    v = bias_regs[cc]
                    for j in range(H):
                        w = lax.bitcast_convert_type(
                            buf[g * LANES + H * bb + j,
                                pl.ds(offs[j] + cc * LANES, LANES)], jnp.int32)
                        v = v + lax.bitcast_convert_type(
                            (w * muls[j]) & jnp.int32(-65536), jnp.float32)
                    out_v[ci * BPC + b, pl.ds(cc * LANES, LANES)] = (
                        jnp.maximum(v, 0.0))
            return _

        lax.fori_loop(0, CHUNK // LANES, grp, 0, unroll=False)

    fire(0, rows_a, sem_a)

    def pair_body(i, _):
        ca = 2 * i
        fire(ca + 1, rows_b, sem_b)
        drain(rows_a, sem_a)
        pool(ca, rows_a)

        @pl.when(i < nchunk // 2 - 1)
        def _fire_next():
            fire(ca + 2, rows_a, sem_a)

        drain(rows_b, sem_b)
        pool(ca + 1, rows_b)
        return _

    lax.fori_loop(0, nchunk // 2, pair_body, 0, unroll=False)
    pltpu.sync_copy(out_v, out_hbm.at[pl.ds(wid * batch_per_w, batch_per_w)])


def _sc_pool(ids_r, p_packed, bias):
    nw, nchunk, chunk = ids_r.shape
    batch_per_w = nchunk * BPC
    batch = nw * batch_per_w
    mesh = plsc.VectorSubcoreMesh(core_axis_name="c", subcore_axis_name="s")
    kern = pl.kernel(
        _sc_pool_body,
        mesh=mesh,
        out_type=jax.ShapeDtypeStruct((batch, D), jnp.float32),
        scratch_types=[
            pltpu.VMEM((nchunk, chunk), jnp.int32),
            pltpu.VMEM((nchunk, chunk), jnp.int32),
            pltpu.VMEM((nchunk, chunk), jnp.int32),
            pltpu.VMEM((D,), jnp.float32),
            pltpu.VMEM((chunk, 2 * D), jnp.float32),
            pltpu.VMEM((chunk, 2 * D), jnp.float32),
            pltpu.VMEM((batch_per_w, D), jnp.float32),
            pltpu.SemaphoreType.DMA,
            pltpu.SemaphoreType.DMA,
        ],
    )
    return kern(ids_r, p_packed, bias)


def kernel(ids, emb_table, proj_w, proj_b):
    batch = ids.shape[0]
    ids_per_w = batch * H // NW
    ids_r = ids.reshape(NW, ids_per_w // CHUNK, CHUNK)
    wt = proj_w.T * (1.0 / H)
    p_packed = _tc_pack(emb_table.T, wt)
    return _sc_pool(ids_r, p_packed, proj_b)
